# speedup vs baseline: 18.7275x; 18.7275x over previous
import functools
import jax
import jax.numpy as jnp
from jax import lax
from jax.experimental import pallas as pl
from jax.experimental.pallas import tpu as pltpu
from jax.experimental.pallas import tpu_sc as plsc

N = 10000
NPAD = 10240
E_ALL = 330000
EPAD = 331776
NC, NS = 2, 16
NW = NC * NS
EPW = EPAD // NW
VPW = EPW // 16
CHUNKS = EPAD // 128 // NS
TROWS = NPAD // NS
BLK = 1024
NBLK = NPAD // BLK

_mesh = plsc.VectorSubcoreMesh(core_axis_name="c", subcore_axis_name="s")
_sc_params = pltpu.CompilerParams(needs_layout_passes=False)


def _dense1_body(x_ref, w_ref, as_ref, ad_ref,
                 htab_ref, es_ref, ed_ref, m_ref, msc):
    i = pl.program_id(0)
    h = jnp.dot(x_ref[...], w_ref[...], preferred_element_type=jnp.float32)
    htab_ref[0] = h[:, :128]
    htab_ref[1] = h[:, 128:]
    es = jnp.sum(h * as_ref[...], axis=1)
    ed = jnp.sum(h * ad_ref[...], axis=1)
    es_ref[...] = es
    ed_ref[...] = ed
    bm = jnp.max(es)

    @pl.when(i == 0)
    def _():
        msc[0] = bm

    @pl.when(i > 0)
    def _():
        msc[0] = jnp.maximum(msc[0], bm)

    @pl.when(i == NBLK - 1)
    def _():
        m_ref[...] = jnp.full((16,), msc[0], jnp.float32)


_dense1 = pl.pallas_call(
    _dense1_body,
    grid=(NBLK,),
    in_specs=[
        pl.BlockSpec((BLK, 128), lambda i: (i, 0)),
        pl.BlockSpec((128, 256), lambda i: (0, 0)),
        pl.BlockSpec((1, 256), lambda i: (0, 0)),
        pl.BlockSpec((1, 256), lambda i: (0, 0)),
    ],
    out_specs=[
        pl.BlockSpec((2, BLK, 128), lambda i: (0, i, 0)),
        pl.BlockSpec((BLK,), lambda i: (i,)),
        pl.BlockSpec((BLK,), lambda i: (i,)),
        pl.BlockSpec((16,), lambda i: (0,)),
    ],
    out_shape=[
        jax.ShapeDtypeStruct((2, NPAD, 128), jnp.float32),
        jax.ShapeDtypeStruct((NPAD,), jnp.float32),
        jax.ShapeDtypeStruct((NPAD,), jnp.float32),
        jax.ShapeDtypeStruct((16,), jnp.float32),
    ],
    scratch_shapes=[pltpu.SMEM((1,), jnp.float32)],
)


def _dense2_body(num_ref, s_ref, b1_ref, w2_ref, as2_ref, ad2_ref,
                 w3_ref, as3_ref, ad3_ref,
                 htab_ref, es2_ref, ed2_ref, m2_ref, es3_ref, ed3_ref, m3_ref,
                 m2sc, m3sc):
    i = pl.program_id(0)
    s = jnp.sum(s_ref[...], axis=0)
    r = 1.0 / (s + 1e-16)
    hcat = jnp.concatenate([num_ref[0] * r[:, None], num_ref[1] * r[:, None]],
                           axis=1) + b1_ref[...]
    h = jnp.maximum(hcat, 0.0)
    h2a = jnp.dot(h, w2_ref[...], preferred_element_type=jnp.float32)
    h2b = jnp.dot(h, w3_ref[...], preferred_element_type=jnp.float32)
    htab_ref[0] = h2a
    htab_ref[1] = h2b
    es2 = jnp.sum(h2a * as2_ref[...], axis=1)
    ed2 = jnp.sum(h2a * ad2_ref[...], axis=1)
    es3 = jnp.sum(h2b * as3_ref[...], axis=1)
    ed3 = jnp.sum(h2b * ad3_ref[...], axis=1)
    es2_ref[...] = es2
    ed2_ref[...] = ed2
    es3_ref[...] = es3
    ed3_ref[...] = ed3
    bm2 = jnp.max(es2)
    bm3 = jnp.max(es3)

    @pl.when(i == 0)
    def _():
        m2sc[0] = bm2
        m3sc[0] = bm3

    @pl.when(i > 0)
    def _():
        m2sc[0] = jnp.maximum(m2sc[0], bm2)
        m3sc[0] = jnp.maximum(m3sc[0], bm3)

    @pl.when(i == NBLK - 1)
    def _():
        m2_ref[...] = jnp.full((16,), m2sc[0], jnp.float32)
        m3_ref[...] = jnp.full((16,), m3sc[0], jnp.float32)


_dense2 = pl.pallas_call(
    _dense2_body,
    grid=(NBLK,),
    in_specs=[
        pl.BlockSpec((2, BLK, 128), lambda i: (0, i, 0)),
        pl.BlockSpec((NW, BLK), lambda i: (0, i)),
        pl.BlockSpec((1, 256), lambda i: (0, 0)),
        pl.BlockSpec((256, 128), lambda i: (0, 0)),
        pl.BlockSpec((1, 128), lambda i: (0, 0)),
        pl.BlockSpec((1, 128), lambda i: (0, 0)),
        pl.BlockSpec((256, 128), lambda i: (0, 0)),
        pl.BlockSpec((1, 128), lambda i: (0, 0)),
        pl.BlockSpec((1, 128), lambda i: (0, 0)),
    ],
    out_specs=[
        pl.BlockSpec((2, BLK, 128), lambda i: (0, i, 0)),
        pl.BlockSpec((BLK,), lambda i: (i,)),
        pl.BlockSpec((BLK,), lambda i: (i,)),
        pl.BlockSpec((16,), lambda i: (0,)),
        pl.BlockSpec((BLK,), lambda i: (i,)),
        pl.BlockSpec((BLK,), lambda i: (i,)),
        pl.BlockSpec((16,), lambda i: (0,)),
    ],
    out_shape=[
        jax.ShapeDtypeStruct((2, NPAD, 128), jnp.float32),
        jax.ShapeDtypeStruct((NPAD,), jnp.float32),
        jax.ShapeDtypeStruct((NPAD,), jnp.float32),
        jax.ShapeDtypeStruct((16,), jnp.float32),
        jax.ShapeDtypeStruct((NPAD,), jnp.float32),
        jax.ShapeDtypeStruct((NPAD,), jnp.float32),
        jax.ShapeDtypeStruct((16,), jnp.float32),
    ],
    scratch_shapes=[pltpu.SMEM((1,), jnp.float32),
                    pltpu.SMEM((1,), jnp.float32)],
)


def _final_body(num_ref, s2_ref, s3_ref, b2_ref, b3_ref, mu_ref, ls_ref):
    s2 = jnp.sum(s2_ref[...], axis=0)
    s3 = jnp.sum(s3_ref[...], axis=0)
    r2 = 1.0 / (s2 + 1e-16)
    r3 = 1.0 / (s3 + 1e-16)
    mu_ref[...] = num_ref[0] * r2[:, None] + b2_ref[...]
    ls_ref[...] = num_ref[1] * r3[:, None] + b3_ref[...]


_final = pl.pallas_call(
    _final_body,
    grid=(NBLK,),
    in_specs=[
        pl.BlockSpec((2, BLK, 128), lambda i: (0, i, 0)),
        pl.BlockSpec((NW, BLK), lambda i: (0, i)),
        pl.BlockSpec((NW, BLK), lambda i: (0, i)),
        pl.BlockSpec((1, 128), lambda i: (0, 0)),
        pl.BlockSpec((1, 128), lambda i: (0, 0)),
    ],
    out_specs=[
        pl.BlockSpec((BLK, 128), lambda i: (i, 0)),
        pl.BlockSpec((BLK, 128), lambda i: (i, 0)),
    ],
    out_shape=[
        jax.ShapeDtypeStruct((NPAD, 128), jnp.float32),
        jax.ShapeDtypeStruct((NPAD, 128), jnp.float32),
    ],
)


def _make_phase_a(L):
    scratch = (
        [pltpu.VMEM((EPW,), jnp.int32), pltpu.VMEM((EPW,), jnp.int32)]
        + [pltpu.VMEM((EPW,), jnp.float32) for _ in range(L)]
        + [pltpu.VMEM((NPAD,), jnp.float32) for _ in range(2 * L)]
        + [pltpu.VMEM((16,), jnp.float32) for _ in range(L)]
        + [pltpu.VMEM((NPAD,), jnp.float32) for _ in range(L)]
    )

    def body(*refs):
        src_h, dst_h = refs[0], refs[1]
        lay = [refs[2 + 3 * l: 5 + 3 * l] for l in range(L)]
        ex_h, s_h = refs[2 + 3 * L], refs[3 + 3 * L]
        sc = list(refs[4 + 3 * L:])
        src_v, dst_v = sc[0], sc[1]
        ex_v = sc[2:2 + L]
        tab_v = sc[2 + L:2 + 3 * L]
        m_v = sc[2 + 3 * L:2 + 4 * L]
        s_v = sc[2 + 4 * L:2 + 5 * L]

        cid = lax.axis_index("c")
        sid = lax.axis_index("s")
        wid = cid * NS + sid
        e0 = wid * EPW
        pltpu.sync_copy(src_h.at[pl.ds(e0, EPW)], src_v)
        pltpu.sync_copy(dst_h.at[pl.ds(e0, EPW)], dst_v)
        for l in range(L):
            pltpu.sync_copy(lay[l][0], tab_v[2 * l])
            pltpu.sync_copy(lay[l][1], tab_v[2 * l + 1])
            pltpu.sync_copy(lay[l][2], m_v[l])

        def zero_body(j, _):
            for l in range(L):
                s_v[l][pl.ds(j * 16, 16)] = jnp.zeros((16,), jnp.float32)
            return 0

        lax.fori_loop(0, NPAD // 16, zero_body, 0)

        lanes = lax.iota(jnp.int32, 16)

        def edge_body(j, _):
            base = j * 16
            isrc = src_v[pl.ds(base, 16)]
            idst = dst_v[pl.ds(base, 16)]
            valid = (e0 + base + lanes) < E_ALL
            for l in range(L):
                a = plsc.load_gather(tab_v[2 * l], [isrc])
                bd = plsc.load_gather(tab_v[2 * l + 1], [idst])
                m = m_v[l][...]
                t = a + bd
                e = jnp.where(t >= 0, t, 0.2 * t)
                tc = bd + m
                cd = jnp.where(tc >= 0, tc, 0.2 * tc)
                exv = jnp.exp(e - cd)
                exv = jnp.where(valid, exv, 0.0)
                ex_v[l][pl.ds(base, 16)] = exv
                plsc.addupdate_scatter(s_v[l], [idst], exv)
            return 0

        lax.fori_loop(0, VPW, edge_body, 0)

        for l in range(L):
            pltpu.sync_copy(ex_v[l], ex_h.at[pl.ds(l * EPAD + e0, EPW)])
            pltpu.sync_copy(s_v[l], s_h.at[l * NW + wid])

    return pl.kernel(
        body,
        out_type=(jax.ShapeDtypeStruct((L * EPAD,), jnp.float32),
                  jax.ShapeDtypeStruct((L * NW, NPAD), jnp.float32)),
        mesh=_mesh,
        compiler_params=_sc_params,
        scratch_types=scratch,
    )


_phase_a1 = _make_phase_a(1)
_phase_a2 = _make_phase_a(2)


def _make_phase_b(percore_ex):
    def body(tab_h, src_h, dstr_h, ex_h, out_h,
             rows_v, dsti_v, srci_v, ex_v, acc_sh, sem):
        cid = lax.axis_index("c")
        sid = lax.axis_index("s")

        def zb(rr, _):
            for q in range(8):
                rows_v[rr, pl.ds(q * 16, 16)] = jnp.zeros((16,), jnp.float32)
            return 0

        lax.fori_loop(0, 128, zb, 0)
        for k in range(TROWS // 128):
            pltpu.sync_copy(rows_v, acc_sh.at[pl.ds(sid * TROWS + k * 128, 128)])
        plsc.subcore_barrier()

        def chunk_body(c, _):
            ch = sid * CHUNKS + c
            eb = ch * 128
            pltpu.sync_copy(src_h.at[pl.ds(cid * EPAD + eb, 128)], srci_v)
            pltpu.sync_copy(dstr_h.at[pl.ds(ch, 1)], dsti_v)
            ex_base = eb + cid * EPAD if percore_ex else eb
            pltpu.sync_copy(ex_h.at[pl.ds(ex_base, 128)], ex_v)
            pltpu.async_copy(tab_h.at[srci_v], rows_v, sem).wait()

            def grp(g, _):
                gb = g * 16
                for e in range(16):
                    ai = jnp.full((16,), gb + e, jnp.int32)
                    ab = plsc.load_gather(ex_v, [ai])
                    row = gb + e
                    for q in range(8):
                        rows_v[row, pl.ds(q * 16, 16)] = (
                            rows_v[row, pl.ds(q * 16, 16)] * ab)
                return 0

            lax.fori_loop(0, 8, grp, 0)
            pltpu.sync_copy(rows_v, acc_sh.at[dsti_v.at[0]], add=True)
            return 0

        lax.fori_loop(0, CHUNKS, chunk_body, 0)
        plsc.subcore_barrier()
        for k in range(TROWS // 128):
            r0 = sid * TROWS + k * 128
            pltpu.sync_copy(acc_sh.at[pl.ds(r0, 128)], rows_v)
            pltpu.sync_copy(rows_v, out_h.at[pl.ds(cid * NPAD + r0, 128)])

    return pl.kernel(
        body,
        out_type=jax.ShapeDtypeStruct((2 * NPAD, 128), jnp.float32),
        mesh=_mesh,
        compiler_params=_sc_params,
        scratch_types=[
            pltpu.VMEM((128, 128), jnp.float32),
            pltpu.VMEM((1, 128), jnp.int32),
            pltpu.VMEM((128,), jnp.int32),
            pltpu.VMEM((128,), jnp.float32),
            pltpu.VMEM_SHARED((NPAD, 128), jnp.float32),
            pltpu.SemaphoreType.DMA,
        ],
    )


_phase_b_shared = _make_phase_b(False)
_phase_b_percore = _make_phase_b(True)


def kernel(x, edge_index, W1, as1, ad1, b1, W2, as2, ad2, b2,
           W3, as3, ad3, b3):
    loop = jnp.arange(N, dtype=edge_index.dtype)
    src = jnp.concatenate([edge_index[0], loop])
    dst = jnp.concatenate([edge_index[1], loop])
    pad = jnp.zeros((EPAD - E_ALL,), edge_index.dtype)
    srcp = jnp.concatenate([src, pad])
    dstp = jnp.concatenate([dst, pad])
    src2 = jnp.concatenate([srcp, srcp + NPAD])
    dstr = dstp.reshape(EPAD // 128, 128)
    xp = jnp.pad(x, ((0, NPAD - N), (0, 0)))

    htab, es1, ed1, m1 = _dense1(xp, W1, as1.reshape(1, -1), ad1.reshape(1, -1))
    ex1, s1 = _phase_a1(srcp, dstp, es1, ed1, m1)
    num1 = _phase_b_shared(htab.reshape(2 * NPAD, 128), src2, dstr, ex1)
    htab2, es2, ed2, m2, es3, ed3, m3 = _dense2(
        num1.reshape(2, NPAD, 128), s1, b1.reshape(1, -1),
        W2, as2.reshape(1, -1), ad2.reshape(1, -1),
        W3, as3.reshape(1, -1), ad3.reshape(1, -1))
    exc, s23 = _phase_a2(srcp, dstp, es2, ed2, m2, es3, ed3, m3)
    num23 = _phase_b_percore(htab2.reshape(2 * NPAD, 128), src2, dstr, exc)
    mu, ls = _final(num23.reshape(2, NPAD, 128), s23[:NW], s23[NW:],
                    b2.reshape(1, -1), b3.reshape(1, -1))
    return mu[:N], ls[:N]
